# Initial kernel scaffold; baseline (speedup 1.0000x reference)
import functools

import jax
import jax.numpy as jnp
from jax import lax
from jax.experimental import pallas as pl
from jax.experimental.pallas import tpu as pltpu
from jax.experimental.pallas import tpu_sc as plsc

N_USERS = 25000
N_NODES = 50000
D = 64
H = 32
N_LAYERS = 2
N_EDGES = 800000

NS = 16
NC = 2
NP = 51200
RT = NP // NS
WB = 800
CH = 1024
EPT = 51200
NCH = EPT // CH
EP = EPT * NS

_mesh = plsc.VectorSubcoreMesh(core_axis_name="c", subcore_axis_name="s")


@functools.partial(
    pl.kernel,
    out_type=[
        jax.ShapeDtypeStruct((NC * NP, H), jnp.float32),
        jax.ShapeDtypeStruct((NC * NP, H), jnp.float32),
    ],
    mesh=_mesh,
    scratch_types=[
        pltpu.VMEM_SHARED((NP, H), jnp.float32),
        pltpu.VMEM((8, 128), jnp.int32),
        pltpu.VMEM((8, 128), jnp.int32),
        pltpu.VMEM((CH,), jnp.float32),
        pltpu.VMEM((CH, H), jnp.float32),
        pltpu.VMEM((WB, H), jnp.float32),
        pltpu.SemaphoreType.DMA,
    ],
)
def _lightgcn_sc(emb2, col2d, row2d, w_hbm, out, ebuf, acc, colv, rowv, wv,
                 gbuf, abuf, sem):
    c = lax.axis_index("c")
    s = lax.axis_index("s")
    coff = c * NP
    row_base = s * RT

    def zero_abuf():
        def body(r, _):
            z = jnp.zeros((16,), jnp.float32)
            abuf[r, 0:16] = z
            abuf[r, 16:32] = z
            return 0
        lax.fori_loop(0, WB, body, 0)

    def layer(src_tab, old_tab, is_last):
        zero_abuf()
        for m in range(RT // WB):
            pltpu.sync_copy(abuf, acc.at[pl.ds(row_base + m * WB, WB)])
        plsc.subcore_barrier()

        def chunk(k, _):
            erow = s * (EPT // 128) + k * (CH // 128)
            ebase = s * EPT + k * CH
            pltpu.sync_copy(col2d.at[pl.ds(erow, 8)], colv)
            pltpu.sync_copy(row2d.at[pl.ds(erow, 8)], rowv)
            pltpu.sync_copy(w_hbm.at[pl.ds(ebase, CH)], wv)
            for j in range(8):
                for i in range(8):
                    colv[j, 16 * i:16 * (i + 1)] = (
                        colv[j, 16 * i:16 * (i + 1)] + coff)
            descs = [
                pltpu.async_copy(src_tab.at[colv.at[j]],
                                 gbuf.at[pl.ds(128 * j, 128)], sem)
                for j in range(8)
            ]
            for d in descs:
                d.wait()

            def scale(r0, _):
                for u in range(8):
                    r = r0 * 8 + u
                    ws = wv[r]
                    gbuf[r, 0:16] = gbuf[r, 0:16] * ws
                    gbuf[r, 16:32] = gbuf[r, 16:32] * ws
                return 0
            lax.fori_loop(0, CH // 8, scale, 0)

            for j in range(8):
                pltpu.sync_copy(gbuf.at[pl.ds(128 * j, 128)],
                                acc.at[rowv.at[j]], add=True)
            return 0

        lax.fori_loop(0, NCH, chunk, 0)
        plsc.subcore_barrier()

        for m in range(RT // WB):
            off = row_base + m * WB
            hoff = coff + off
            pltpu.sync_copy(acc.at[pl.ds(off, WB)], gbuf.at[pl.ds(0, WB)])
            pltpu.sync_copy(old_tab.at[pl.ds(hoff, WB)], abuf)

            def accum(r, _):
                a0 = abuf[r, 0:16] + gbuf[r, 0:16]
                a1 = abuf[r, 16:32] + gbuf[r, 16:32]
                if is_last:
                    third = jnp.float32(1.0 / 3.0)
                    a0 = a0 * third
                    a1 = a1 * third
                abuf[r, 0:16] = a0
                abuf[r, 16:32] = a1
                return 0
            lax.fori_loop(0, WB, accum, 0)

            pltpu.sync_copy(abuf, out.at[pl.ds(hoff, WB)])
            if not is_last:
                pltpu.sync_copy(gbuf.at[pl.ds(0, WB)],
                                ebuf.at[pl.ds(hoff, WB)])
        plsc.subcore_barrier()

    layer(emb2, emb2, is_last=False)
    layer(ebuf, out, is_last=True)


def kernel(embedding, edge_weight, edge_index):
    row = edge_index[0].astype(jnp.int32)
    col = edge_index[1].astype(jnp.int32)
    w = edge_weight.astype(jnp.float32)

    emb2 = jnp.zeros((NC, NP, H), jnp.float32)
    emb2 = emb2.at[0, :N_NODES].set(embedding[:, :H])
    emb2 = emb2.at[1, :N_NODES].set(embedding[:, H:])
    emb2 = emb2.reshape(NC * NP, H)

    colp = jnp.zeros((EP,), jnp.int32).at[:N_EDGES].set(col).reshape(EP // 128, 128)
    rowp = jnp.full((EP,), N_NODES, jnp.int32).at[:N_EDGES].set(row).reshape(EP // 128, 128)
    wp = jnp.zeros((EP,), jnp.float32).at[:N_EDGES].set(w)

    out, _ = _lightgcn_sc(emb2, colp, rowp, wp)
    halves = out.reshape(NC, NP, H)[:, :N_NODES, :]
    e_final = jnp.concatenate([halves[0], halves[1]], axis=1)
    return (e_final[:N_USERS], e_final[N_USERS:])


# speedup vs baseline: 5.7325x; 5.7325x over previous
import functools

import jax
import jax.numpy as jnp
from jax import lax
from jax.experimental import pallas as pl
from jax.experimental.pallas import tpu as pltpu
from jax.experimental.pallas import tpu_sc as plsc

N_USERS = 25000
N_NODES = 50000
H = 32

NS = 16
NC = 2
NACC = 50008
RT = N_NODES // NS
WB = 125
NWB = RT // WB
CH = 512
EPT = 50176
NCH = EPT // CH
EP = EPT * NS
N_EDGES = 800000

_mesh = plsc.VectorSubcoreMesh(core_axis_name="c", subcore_axis_name="s")


@functools.partial(
    pl.kernel,
    out_type=[
        jax.ShapeDtypeStruct((NC * N_NODES, H), jnp.float32),
        jax.ShapeDtypeStruct((NC * N_NODES, H), jnp.float32),
    ],
    mesh=_mesh,
    scratch_types=[
        pltpu.VMEM_SHARED((NACC, H), jnp.float32),
        pltpu.VMEM((4, 128), jnp.int32),
        pltpu.VMEM((4, 128), jnp.int32),
        pltpu.VMEM((CH,), jnp.float32),
        pltpu.VMEM((CH, H), jnp.float32),
        pltpu.VMEM((WB, H), jnp.float32),
        pltpu.SemaphoreType.DMA,
    ],
    compiler_params=pltpu.CompilerParams(use_tc_tiling_on_sc=False),
)
def _lightgcn_sc(emb2, col2d, row2d, w_hbm, out, ebuf, acc, colv, rowv, wv,
                 gbuf, abuf, sem):
    c = lax.axis_index("c")
    s = lax.axis_index("s")
    coff = c * N_NODES
    row_base = s * RT

    def layer(src_tab, old_tab, is_last):
        def zbody(r, _):
            z = jnp.zeros((16,), jnp.float32)
            abuf[r, 0:16] = z
            abuf[r, 16:32] = z
            return 0
        lax.fori_loop(0, WB, zbody, 0)
        for m in range(NWB):
            pltpu.sync_copy(abuf, acc.at[pl.ds(row_base + m * WB, WB)])

        @pl.when(s == 0)
        def _():
            pltpu.sync_copy(abuf.at[pl.ds(0, 8)],
                            acc.at[pl.ds(N_NODES, 8)])
        plsc.subcore_barrier()

        def chunk(k, _):
            erow = s * (EPT // 128) + k * (CH // 128)
            ebase = s * EPT + k * CH
            pltpu.sync_copy(col2d.at[pl.ds(erow, 4)], colv)
            pltpu.sync_copy(row2d.at[pl.ds(erow, 4)], rowv)
            pltpu.sync_copy(w_hbm.at[pl.ds(ebase, CH)], wv)
            for j in range(4):
                for i in range(8):
                    colv[j, 16 * i:16 * (i + 1)] = (
                        colv[j, 16 * i:16 * (i + 1)] + coff)
            descs = [
                pltpu.async_copy(src_tab.at[colv.at[j]],
                                 gbuf.at[pl.ds(128 * j, 128)], sem)
                for j in range(4)
            ]
            for d in descs:
                d.wait()

            def scale(g, _):
                wvec = wv[pl.ds(g * 16, 16)]
                for u in range(16):
                    r = g * 16 + u
                    ws = wvec[u]
                    gbuf[r, 0:16] = gbuf[r, 0:16] * ws
                    gbuf[r, 16:32] = gbuf[r, 16:32] * ws
                return 0
            lax.fori_loop(0, CH // 16, scale, 0)

            for j in range(4):
                pltpu.sync_copy(gbuf.at[pl.ds(128 * j, 128)],
                                acc.at[rowv.at[j]], add=True)
            return 0

        lax.fori_loop(0, NCH, chunk, 0)
        plsc.subcore_barrier()

        for m in range(NWB):
            off = row_base + m * WB
            hoff = coff + off
            pltpu.sync_copy(acc.at[pl.ds(off, WB)], gbuf.at[pl.ds(0, WB)])
            pltpu.sync_copy(old_tab.at[pl.ds(hoff, WB)], abuf)

            def accum(r, _):
                a0 = abuf[r, 0:16] + gbuf[r, 0:16]
                a1 = abuf[r, 16:32] + gbuf[r, 16:32]
                if is_last:
                    third = jnp.float32(1.0 / 3.0)
                    a0 = a0 * third
                    a1 = a1 * third
                abuf[r, 0:16] = a0
                abuf[r, 16:32] = a1
                return 0
            lax.fori_loop(0, WB, accum, 0)

            pltpu.sync_copy(abuf, out.at[pl.ds(hoff, WB)])
            if not is_last:
                pltpu.sync_copy(gbuf.at[pl.ds(0, WB)],
                                ebuf.at[pl.ds(hoff, WB)])
        plsc.subcore_barrier()

    layer(emb2, emb2, is_last=False)
    layer(ebuf, out, is_last=True)


def kernel(embedding, edge_weight, edge_index):
    row = edge_index[0].astype(jnp.int32)
    col = edge_index[1].astype(jnp.int32)
    w = edge_weight.astype(jnp.float32)

    emb2 = jnp.stack([embedding[:, :H], embedding[:, H:]], axis=0)
    emb2 = emb2.reshape(NC * N_NODES, H)

    colp = jnp.zeros((EP,), jnp.int32).at[:N_EDGES].set(col).reshape(EP // 128, 128)
    rowp = jnp.full((EP,), N_NODES, jnp.int32).at[:N_EDGES].set(row).reshape(EP // 128, 128)
    wp = jnp.zeros((EP,), jnp.float32).at[:N_EDGES].set(w)

    out, _ = _lightgcn_sc(emb2, colp, rowp, wp)
    halves = out.reshape(NC, N_NODES, H)
    e_final = jnp.concatenate([halves[0], halves[1]], axis=1)
    return (e_final[:N_USERS], e_final[N_USERS:])


# speedup vs baseline: 5.7793x; 1.0082x over previous
import functools

import jax
import jax.numpy as jnp
from jax import lax
from jax.experimental import pallas as pl
from jax.experimental.pallas import tpu as pltpu
from jax.experimental.pallas import tpu_sc as plsc

N_USERS = 25000
N_NODES = 50000
H = 32

NS = 16
NC = 2
NACC = 50008
RT = N_NODES // NS
WB = 125
NWB = RT // WB
CH = 256
NPC = 8
SUP = CH * NPC
EPT = 51200
NSUP = EPT // SUP
EP = EPT * NS
N_EDGES = 800000

_mesh = plsc.VectorSubcoreMesh(core_axis_name="c", subcore_axis_name="s")


@functools.partial(
    pl.kernel,
    out_type=[
        jax.ShapeDtypeStruct((NC * N_NODES, H), jnp.float32),
        jax.ShapeDtypeStruct((NC * N_NODES, H), jnp.float32),
    ],
    mesh=_mesh,
    scratch_types=[
        pltpu.VMEM_SHARED((NACC, H), jnp.float32),
        pltpu.VMEM((SUP // 128, 128), jnp.int32),
        pltpu.VMEM((SUP // 128, 128), jnp.int32),
        pltpu.VMEM((SUP,), jnp.float32),
        pltpu.VMEM((2, CH, H), jnp.float32),
        pltpu.VMEM((WB, H), jnp.float32),
        pltpu.SemaphoreType.DMA,
        pltpu.SemaphoreType.DMA,
    ],
    compiler_params=pltpu.CompilerParams(use_tc_tiling_on_sc=False),
)
def _lightgcn_sc(emb2, col2d, row2d, w_hbm, out, ebuf, acc, colv, rowv, wv,
                 gbuf, abuf, sem0, sem1):
    c = lax.axis_index("c")
    s = lax.axis_index("s")
    coff = c * N_NODES
    row_base = s * RT
    sems = (sem0, sem1)

    def layer(src_tab, old_tab, is_last):
        def zbody(r, _):
            z = jnp.zeros((16,), jnp.float32)
            abuf[r, 0:16] = z
            abuf[r, 16:32] = z
            return 0
        lax.fori_loop(0, WB, zbody, 0)
        for m in range(NWB):
            pltpu.sync_copy(abuf, acc.at[pl.ds(row_base + m * WB, WB)])

        @pl.when(s == 0)
        def _():
            pltpu.sync_copy(abuf.at[pl.ds(0, 8)], acc.at[pl.ds(N_NODES, 8)])
        plsc.subcore_barrier()

        def sup_body(t, _):
            erow = s * (EPT // 128) + t * (SUP // 128)
            ebase = s * EPT + t * SUP
            pltpu.sync_copy(col2d.at[pl.ds(erow, SUP // 128)], colv)
            pltpu.sync_copy(row2d.at[pl.ds(erow, SUP // 128)], rowv)
            pltpu.sync_copy(w_hbm.at[pl.ds(ebase, SUP)], wv)
            def cadd(j, _):
                for i in range(8):
                    colv[j, 16 * i:16 * (i + 1)] = (
                        colv[j, 16 * i:16 * (i + 1)] + coff)
                return 0
            lax.fori_loop(0, SUP // 128, cadd, 0)

            def fire(cc):
                slot = cc % 2
                return [
                    pltpu.async_copy(
                        src_tab.at[colv.at[2 * cc + j]],
                        gbuf.at[slot].at[pl.ds(128 * j, 128)], sems[slot])
                    for j in range(2)
                ]

            descs = fire(0)
            for cc in range(NPC):
                slot = cc % 2
                nxt = fire(cc + 1) if cc + 1 < NPC else None
                for d in descs:
                    d.wait()
                descs = nxt

                def scale(g, _):
                    wvec = wv[pl.ds(cc * CH + g * 16, 16)]
                    for u in range(16):
                        r = g * 16 + u
                        ws = wvec.at[jnp.full((16,), u, jnp.int32)].get(
                            mode="promise_in_bounds")
                        gbuf[slot, r, 0:16] = gbuf[slot, r, 0:16] * ws
                        gbuf[slot, r, 16:32] = gbuf[slot, r, 16:32] * ws
                    return 0
                lax.fori_loop(0, CH // 16, scale, 0)

                for j in range(2):
                    pltpu.sync_copy(gbuf.at[slot].at[pl.ds(128 * j, 128)],
                                    acc.at[rowv.at[2 * cc + j]], add=True)
            return 0

        lax.fori_loop(0, NSUP, sup_body, 0)
        plsc.subcore_barrier()

        for m in range(NWB):
            off = row_base + m * WB
            hoff = coff + off
            pltpu.sync_copy(acc.at[pl.ds(off, WB)],
                            gbuf.at[0].at[pl.ds(0, WB)])
            pltpu.sync_copy(old_tab.at[pl.ds(hoff, WB)], abuf)

            def accum(r, _):
                a0 = abuf[r, 0:16] + gbuf[0, r, 0:16]
                a1 = abuf[r, 16:32] + gbuf[0, r, 16:32]
                if is_last:
                    third = jnp.float32(1.0 / 3.0)
                    a0 = a0 * third
                    a1 = a1 * third
                abuf[r, 0:16] = a0
                abuf[r, 16:32] = a1
                return 0
            lax.fori_loop(0, WB, accum, 0)

            pltpu.sync_copy(abuf, out.at[pl.ds(hoff, WB)])
            if not is_last:
                pltpu.sync_copy(gbuf.at[0].at[pl.ds(0, WB)],
                                ebuf.at[pl.ds(hoff, WB)])
        plsc.subcore_barrier()

    layer(emb2, emb2, is_last=False)
    layer(ebuf, out, is_last=True)


def kernel(embedding, edge_weight, edge_index):
    row = edge_index[0].astype(jnp.int32)
    col = edge_index[1].astype(jnp.int32)
    w = edge_weight.astype(jnp.float32)

    emb2 = jnp.stack([embedding[:, :H], embedding[:, H:]], axis=0)
    emb2 = emb2.reshape(NC * N_NODES, H)

    colp = jnp.zeros((EP,), jnp.int32).at[:N_EDGES].set(col).reshape(EP // 128, 128)
    rowp = jnp.full((EP,), N_NODES, jnp.int32).at[:N_EDGES].set(row).reshape(EP // 128, 128)
    wp = jnp.zeros((EP,), jnp.float32).at[:N_EDGES].set(w)

    out, _ = _lightgcn_sc(emb2, colp, rowp, wp)
    halves = out.reshape(NC, N_NODES, H)
    e_final = jnp.concatenate([halves[0], halves[1]], axis=1)
    return (e_final[:N_USERS], e_final[N_USERS:])


# speedup vs baseline: 6.0439x; 1.0458x over previous
import functools

import jax
import jax.numpy as jnp
from jax import lax
from jax.experimental import pallas as pl
from jax.experimental.pallas import tpu as pltpu
from jax.experimental.pallas import tpu_sc as plsc

N_USERS = 25000
N_NODES = 50000
H = 32

NS = 16
NC = 2
NACC = 50008
RT = N_NODES // NS
WB = 125
NWB = RT // WB
CH = 256
NPC = 8
SUP = CH * NPC
EPT = 51200
NSUP = EPT // SUP
EP = EPT * NS
N_EDGES = 800000

_mesh = plsc.VectorSubcoreMesh(core_axis_name="c", subcore_axis_name="s")


@functools.partial(
    pl.kernel,
    out_type=[
        jax.ShapeDtypeStruct((NC * N_NODES, H), jnp.float32),
        jax.ShapeDtypeStruct((NC * N_NODES, H), jnp.float32),
    ],
    mesh=_mesh,
    scratch_types=[
        pltpu.VMEM_SHARED((NACC, H), jnp.float32),
        pltpu.VMEM((SUP // 128, 128), jnp.int32),
        pltpu.VMEM((SUP // 128, 128), jnp.int32),
        pltpu.VMEM((SUP,), jnp.float32),
        pltpu.VMEM((2, CH, H), jnp.float32),
        pltpu.VMEM((WB, H), jnp.float32),
        pltpu.SemaphoreType.DMA,
        pltpu.SemaphoreType.DMA,
        pltpu.SemaphoreType.DMA,
        pltpu.SemaphoreType.DMA,
    ],
    compiler_params=pltpu.CompilerParams(use_tc_tiling_on_sc=False),
)
def _lightgcn_sc(emb2, col2d, row2d, w_hbm, out, ebuf, acc, colv, rowv, wv,
                 gbuf, abuf, sem0, sem1, sem_i, sem_s):
    c = lax.axis_index("c")
    s = lax.axis_index("s")
    coff = c * N_NODES
    row_base = s * RT
    sems = (sem0, sem1)

    def layer(src_tab, old_tab, is_last):
        def zbody(r, _):
            z = jnp.zeros((16,), jnp.float32)
            abuf[r, 0:16] = z
            abuf[r, 16:32] = z
            return 0
        lax.fori_loop(0, WB, zbody, 0)
        for m in range(NWB):
            pltpu.sync_copy(abuf, acc.at[pl.ds(row_base + m * WB, WB)])

        @pl.when(s == 0)
        def _():
            pltpu.sync_copy(abuf.at[pl.ds(0, 8)], acc.at[pl.ds(N_NODES, 8)])
        plsc.subcore_barrier()

        def sup_body(t, _):
            erow = s * (EPT // 128) + t * (SUP // 128)
            ebase = s * EPT + t * SUP
            idx_descs = [
                pltpu.async_copy(col2d.at[pl.ds(erow, SUP // 128)], colv,
                                 sem_i),
                pltpu.async_copy(row2d.at[pl.ds(erow, SUP // 128)], rowv,
                                 sem_i),
                pltpu.async_copy(w_hbm.at[pl.ds(ebase, SUP)], wv, sem_i),
            ]
            for d in idx_descs:
                d.wait()
            def cadd(j, _):
                for i in range(8):
                    colv[j, 16 * i:16 * (i + 1)] = (
                        colv[j, 16 * i:16 * (i + 1)] + coff)
                return 0
            lax.fori_loop(0, SUP // 128, cadd, 0)

            def fire(cc):
                slot = cc % 2
                return [
                    pltpu.async_copy(
                        src_tab.at[colv.at[2 * cc + j]],
                        gbuf.at[slot].at[pl.ds(128 * j, 128)], sems[slot])
                    for j in range(2)
                ]

            descs = fire(0)
            sc_prev = None
            for cc in range(NPC):
                slot = cc % 2
                if sc_prev is not None:
                    for d in sc_prev:
                        d.wait()
                nxt = fire(cc + 1) if cc + 1 < NPC else None
                for d in descs:
                    d.wait()
                descs = nxt

                def scale(g, _):
                    wvec = wv[pl.ds(cc * CH + g * 16, 16)]
                    for u in range(16):
                        r = g * 16 + u
                        ws = wvec.at[jnp.full((16,), u, jnp.int32)].get(
                            mode="promise_in_bounds")
                        gbuf[slot, r, 0:16] = gbuf[slot, r, 0:16] * ws
                        gbuf[slot, r, 16:32] = gbuf[slot, r, 16:32] * ws
                    return 0
                lax.fori_loop(0, CH // 16, scale, 0)

                sc_prev = [
                    pltpu.async_copy(gbuf.at[slot].at[pl.ds(128 * j, 128)],
                                     acc.at[rowv.at[2 * cc + j]], sem_s,
                                     add=True)
                    for j in range(2)
                ]
            for d in sc_prev:
                d.wait()
            return 0

        lax.fori_loop(0, NSUP, sup_body, 0)
        plsc.subcore_barrier()

        for m in range(NWB):
            off = row_base + m * WB
            hoff = coff + off
            pltpu.sync_copy(acc.at[pl.ds(off, WB)],
                            gbuf.at[0].at[pl.ds(0, WB)])
            pltpu.sync_copy(old_tab.at[pl.ds(hoff, WB)], abuf)

            def accum(r, _):
                a0 = abuf[r, 0:16] + gbuf[0, r, 0:16]
                a1 = abuf[r, 16:32] + gbuf[0, r, 16:32]
                if is_last:
                    third = jnp.float32(1.0 / 3.0)
                    a0 = a0 * third
                    a1 = a1 * third
                abuf[r, 0:16] = a0
                abuf[r, 16:32] = a1
                return 0
            lax.fori_loop(0, WB, accum, 0)

            pltpu.sync_copy(abuf, out.at[pl.ds(hoff, WB)])
            if not is_last:
                pltpu.sync_copy(gbuf.at[0].at[pl.ds(0, WB)],
                                ebuf.at[pl.ds(hoff, WB)])
        plsc.subcore_barrier()

    layer(emb2, emb2, is_last=False)
    layer(ebuf, out, is_last=True)


def kernel(embedding, edge_weight, edge_index):
    row = edge_index[0].astype(jnp.int32)
    col = edge_index[1].astype(jnp.int32)
    w = edge_weight.astype(jnp.float32)

    emb2 = jnp.stack([embedding[:, :H], embedding[:, H:]], axis=0)
    emb2 = emb2.reshape(NC * N_NODES, H)

    colp = jnp.zeros((EP,), jnp.int32).at[:N_EDGES].set(col).reshape(EP // 128, 128)
    rowp = jnp.full((EP,), N_NODES, jnp.int32).at[:N_EDGES].set(row).reshape(EP // 128, 128)
    wp = jnp.zeros((EP,), jnp.float32).at[:N_EDGES].set(w)

    out, _ = _lightgcn_sc(emb2, colp, rowp, wp)
    halves = out.reshape(NC, N_NODES, H)
    e_final = jnp.concatenate([halves[0], halves[1]], axis=1)
    return (e_final[:N_USERS], e_final[N_USERS:])
